# baseline (speedup 1.0000x reference)
import functools

import jax
import jax.numpy as jnp
from jax import lax
from jax.experimental import pallas as pl
from jax.experimental.pallas import tpu as pltpu
from jax.experimental.pallas import tpu_sc as plsc

N = 2048
D = 768
H = 1536
E = 8
EMBED = 768

BLK = 256
NB = N * 2 // BLK + E
P = NB * BLK
NW = 32
TPW = N // NW


def _route_body(x_ref, wg_ref, pos0_ref, pos1_ref, g0_ref, g1_ref, be_ref,
                nbt_ref):
    logits = jnp.dot(x_ref[...], wg_ref[...])
    ioE = lax.broadcasted_iota(jnp.int32, (N, E), 1)
    m1 = jnp.max(logits, axis=1, keepdims=True)
    a1 = jnp.min(jnp.where(logits == m1, ioE, E), axis=1, keepdims=True)
    masked = jnp.where(ioE == a1, -jnp.inf, logits)
    m2 = jnp.max(masked, axis=1, keepdims=True)
    a2 = jnp.min(jnp.where(masked == m2, ioE, E), axis=1, keepdims=True)
    t = jnp.exp(m2 - m1)
    gtop = 1.0 / (1.0 + t)
    gsec = t / (1.0 + t)
    ones16 = jnp.ones((1, 16), jnp.float32)
    g0_ref[...] = gtop * ones16
    g1_ref[...] = gsec * ones16

    oh0 = (ioE == a1).astype(jnp.float32)
    oh1 = (ioE == a2).astype(jnp.float32)

    CH = 256
    ior = lax.broadcasted_iota(jnp.int32, (CH, CH), 0)
    ioc = lax.broadcasted_iota(jnp.int32, (CH, CH), 1)
    tri = (ior >= ioc).astype(jnp.float32)

    def chunked_cumsum(oh):
        tot = jnp.zeros((1, E), jnp.float32)
        parts = []
        for i in range(N // CH):
            c = jnp.dot(tri, oh[i * CH:(i + 1) * CH, :]) + tot
            parts.append(c)
            tot = c[CH - 1:CH, :]
        return jnp.concatenate(parts, axis=0), tot

    C0, cnt0 = chunked_cumsum(oh0)
    C1, cnt1 = chunked_cumsum(oh1)
    counts = cnt0 + cnt1
    nblk = jnp.floor((counts + (BLK - 1)) * (1.0 / BLK))
    i8r = lax.broadcasted_iota(jnp.int32, (E, E), 0)
    i8c = lax.broadcasted_iota(jnp.int32, (E, E), 1)
    triu = (i8r <= i8c).astype(jnp.float32)
    cumb = jnp.dot(nblk, triu)
    cumb_ex = cumb - nblk
    offs = BLK * cumb_ex
    pos0f = jnp.sum(oh0 * (offs + C0 - oh0), axis=1, keepdims=True)
    pos1f = jnp.sum(oh1 * (offs + cnt0 + C1 - oh1), axis=1, keepdims=True)
    pos0_ref[...] = pos0f.astype(jnp.int32)
    pos1_ref[...] = pos1f.astype(jnp.int32)

    nbt = jnp.sum(nblk, axis=1, keepdims=True)
    iob = lax.broadcasted_iota(jnp.int32, (NB, E), 0).astype(jnp.float32)
    iob = jnp.minimum(iob, nbt - 1.0)
    be = jnp.sum((iob >= cumb_ex).astype(jnp.float32), axis=1,
                 keepdims=True) - 1.0
    be_ref[...] = be.astype(jnp.int32)
    nbt_ref[...] = nbt.astype(jnp.int32)


def _route(x, w_gate):
    return pl.pallas_call(
        _route_body,
        out_shape=[
            jax.ShapeDtypeStruct((N, 1), jnp.int32),
            jax.ShapeDtypeStruct((N, 1), jnp.int32),
            jax.ShapeDtypeStruct((N, 16), jnp.float32),
            jax.ShapeDtypeStruct((N, 16), jnp.float32),
            jax.ShapeDtypeStruct((NB, 1), jnp.int32),
            jax.ShapeDtypeStruct((1, 1), jnp.int32),
        ],
    )(x, w_gate)


_SC_CACHE = {}


def _dispatch_body(x_hbm, pos0_hbm, pos1_hbm, xs_hbm,
                   rows_v, idx0_v, idx1_v, sem):
    wid = lax.axis_index("s") * 2 + lax.axis_index("c")
    base = wid * TPW
    pltpu.sync_copy(x_hbm.at[pl.ds(base, TPW)], rows_v)
    pltpu.sync_copy(pos0_hbm.at[pl.ds(base, TPW)], idx0_v)
    pltpu.sync_copy(pos1_hbm.at[pl.ds(base, TPW)], idx1_v)
    c0 = pltpu.async_copy(rows_v, xs_hbm.at[idx0_v], sem)
    c1 = pltpu.async_copy(rows_v, xs_hbm.at[idx1_v], sem)
    c0.wait()
    c1.wait()


def _dispatch(x, p0f, p1f):
    if "dispatch" not in _SC_CACHE:
        mesh = plsc.VectorSubcoreMesh(core_axis_name="c", subcore_axis_name="s")
        _SC_CACHE["dispatch"] = functools.partial(
            pl.kernel, mesh=mesh,
            out_type=jax.ShapeDtypeStruct((P, D), jnp.float32),
            scratch_types=[
                pltpu.VMEM((TPW, D), jnp.float32),
                pltpu.VMEM((TPW,), jnp.int32),
                pltpu.VMEM((TPW,), jnp.int32),
                pltpu.SemaphoreType.DMA,
            ])(_dispatch_body)
    return _SC_CACHE["dispatch"](x, p0f, p1f)


def _ffn_body(be_ref, nbt_ref, xs_ref, W1_ref, b1_ref, W2_ref, b2_ref, ys_ref):
    b = pl.program_id(0)

    @pl.when(b < nbt_ref[0])
    def _compute():
        xb = xs_ref[...].astype(jnp.bfloat16)
        h = jax.nn.gelu(
            jnp.dot(xb, W1_ref[0],
                    preferred_element_type=jnp.float32) + b1_ref[0])
        ys_ref[...] = jnp.dot(h.astype(jnp.bfloat16), W2_ref[0],
                              preferred_element_type=jnp.float32) + b2_ref[0]


def _ffn(be, nbt, xs, W1b, b1, W2b, b2):
    grid_spec = pltpu.PrefetchScalarGridSpec(
        num_scalar_prefetch=2,
        grid=(NB,),
        in_specs=[
            pl.BlockSpec((BLK, D), lambda b, ber, nbr: (b, 0)),
            pl.BlockSpec((1, D, H), lambda b, ber, nbr: (ber[b], 0, 0)),
            pl.BlockSpec((1, 1, H), lambda b, ber, nbr: (ber[b], 0, 0)),
            pl.BlockSpec((1, H, D), lambda b, ber, nbr: (ber[b], 0, 0)),
            pl.BlockSpec((1, 1, D), lambda b, ber, nbr: (ber[b], 0, 0)),
        ],
        out_specs=pl.BlockSpec((BLK, D), lambda b, ber, nbr: (b, 0)),
    )
    return pl.pallas_call(
        _ffn_body,
        grid_spec=grid_spec,
        out_shape=jax.ShapeDtypeStruct((P, D), jnp.float32),
        compiler_params=pltpu.CompilerParams(
            dimension_semantics=("arbitrary",)),
    )(be, nbt, xs, W1b, b1, W2b, b2)


def _combine_body(ys_hbm, pos0_hbm, pos1_hbm, y0_hbm, y1_hbm,
                  idx0_v, idx1_v, a_v, b_v, sem):
    wid = lax.axis_index("s") * 2 + lax.axis_index("c")
    base = wid * TPW
    pltpu.sync_copy(pos0_hbm.at[pl.ds(base, TPW)], idx0_v)
    pltpu.sync_copy(pos1_hbm.at[pl.ds(base, TPW)], idx1_v)
    c0 = pltpu.async_copy(ys_hbm.at[idx0_v], a_v, sem)
    c1 = pltpu.async_copy(ys_hbm.at[idx1_v], b_v, sem)
    c0.wait()
    pltpu.sync_copy(a_v, y0_hbm.at[pl.ds(base, TPW)])
    c1.wait()
    pltpu.sync_copy(b_v, y1_hbm.at[pl.ds(base, TPW)])


def _combine(ys, p0f, p1f):
    if "combine" not in _SC_CACHE:
        mesh = plsc.VectorSubcoreMesh(core_axis_name="c", subcore_axis_name="s")
        _SC_CACHE["combine"] = functools.partial(
            pl.kernel, mesh=mesh,
            out_type=[jax.ShapeDtypeStruct((N, D), jnp.float32),
                      jax.ShapeDtypeStruct((N, D), jnp.float32)],
            scratch_types=[
                pltpu.VMEM((TPW,), jnp.int32),
                pltpu.VMEM((TPW,), jnp.int32),
                pltpu.VMEM((TPW, D), jnp.float32),
                pltpu.VMEM((TPW, D), jnp.float32),
                pltpu.SemaphoreType.DMA,
            ])(_combine_body)
    return _SC_CACHE["combine"](ys, p0f, p1f)


def _proj_body(y0_ref, y1_ref, g0_ref, g1_ref, pw_ref, pb_ref, out_ref):
    y = g0_ref[...] * y0_ref[...] + g1_ref[...] * y1_ref[...]
    out_ref[...] = lax.dot_general(
        y, pw_ref[...], (((1,), (1,)), ((), ())),
        precision=lax.Precision.DEFAULT) + pb_ref[...]


def _proj(y0, y1, g0c, g1c, proj_w, proj_b2d):
    return pl.pallas_call(
        _proj_body,
        grid=(N // 256,),
        in_specs=[
            pl.BlockSpec((256, D), lambda i: (i, 0)),
            pl.BlockSpec((256, D), lambda i: (i, 0)),
            pl.BlockSpec((256, 1), lambda i: (i, 0)),
            pl.BlockSpec((256, 1), lambda i: (i, 0)),
            pl.BlockSpec((EMBED, D), lambda i: (0, 0)),
            pl.BlockSpec((1, EMBED), lambda i: (0, 0)),
        ],
        out_specs=pl.BlockSpec((256, EMBED), lambda i: (i, 0)),
        out_shape=jax.ShapeDtypeStruct((N, EMBED), jnp.float32),
        compiler_params=pltpu.CompilerParams(
            dimension_semantics=("parallel",)),
    )(y0, y1, g0c, g1c, proj_w, proj_b2d)


def kernel(x, w_gate, W1, b1, W2, b2, proj_w, proj_b):
    pos0, pos1, g0b, g1b, be, nbt = _route(x, w_gate)
    p0f = pos0.reshape(N)
    p1f = pos1.reshape(N)
    xs = _dispatch(x, p0f, p1f)
    ys = _ffn(be.reshape(NB), nbt.reshape(1),
              xs, W1.astype(jnp.bfloat16), b1.reshape(E, 1, H),
              W2.astype(jnp.bfloat16), b2.reshape(E, 1, D))
    y0, y1 = _combine(ys, p0f, p1f)
    return _proj(y0, y1, g0b[:, :1], g1b[:, :1],
                 proj_w, proj_b.reshape(1, EMBED))


# speedup vs baseline: 1.2332x; 1.2332x over previous
import functools

import jax
import jax.numpy as jnp
from jax import lax
from jax.experimental import pallas as pl
from jax.experimental.pallas import tpu as pltpu
from jax.experimental.pallas import tpu_sc as plsc

N = 2048
D = 768
H = 1536
E = 8
EMBED = 768

BLK = 256
NB = N * 2 // BLK + E
P = NB * BLK
NW = 32
TPW = N // NW


def _route_body(x_ref, wg_ref, pos0_ref, pos1_ref, g0_ref, g1_ref, be_ref,
                nbt_ref):
    logits = jnp.dot(x_ref[...], wg_ref[...])
    ioE = lax.broadcasted_iota(jnp.int32, (N, E), 1)
    m1 = jnp.max(logits, axis=1, keepdims=True)
    a1 = jnp.min(jnp.where(logits == m1, ioE, E), axis=1, keepdims=True)
    masked = jnp.where(ioE == a1, -jnp.inf, logits)
    m2 = jnp.max(masked, axis=1, keepdims=True)
    a2 = jnp.min(jnp.where(masked == m2, ioE, E), axis=1, keepdims=True)
    t = jnp.exp(m2 - m1)
    gtop = 1.0 / (1.0 + t)
    gsec = t / (1.0 + t)
    ones16 = jnp.ones((1, 16), jnp.float32)
    g0_ref[...] = gtop * ones16
    g1_ref[...] = gsec * ones16

    oh0 = (ioE == a1).astype(jnp.float32)
    oh1 = (ioE == a2).astype(jnp.float32)

    CH = 256
    ior = lax.broadcasted_iota(jnp.int32, (CH, CH), 0)
    ioc = lax.broadcasted_iota(jnp.int32, (CH, CH), 1)
    tri = (ior >= ioc).astype(jnp.float32)

    def chunked_cumsum(oh):
        tot = jnp.zeros((1, E), jnp.float32)
        parts = []
        for i in range(N // CH):
            c = jnp.dot(tri, oh[i * CH:(i + 1) * CH, :]) + tot
            parts.append(c)
            tot = c[CH - 1:CH, :]
        return jnp.concatenate(parts, axis=0), tot

    C0, cnt0 = chunked_cumsum(oh0)
    C1, cnt1 = chunked_cumsum(oh1)
    counts = cnt0 + cnt1
    nblk = jnp.floor((counts + (BLK - 1)) * (1.0 / BLK))
    i8r = lax.broadcasted_iota(jnp.int32, (E, E), 0)
    i8c = lax.broadcasted_iota(jnp.int32, (E, E), 1)
    triu = (i8r <= i8c).astype(jnp.float32)
    cumb = jnp.dot(nblk, triu)
    cumb_ex = cumb - nblk
    offs = BLK * cumb_ex
    pos0f = jnp.sum(oh0 * (offs + C0 - oh0), axis=1, keepdims=True)
    pos1f = jnp.sum(oh1 * (offs + cnt0 + C1 - oh1), axis=1, keepdims=True)
    pos0_ref[...] = pos0f.astype(jnp.int32)
    pos1_ref[...] = pos1f.astype(jnp.int32)

    nbt = jnp.sum(nblk, axis=1, keepdims=True)
    iob = lax.broadcasted_iota(jnp.int32, (NB, E), 0).astype(jnp.float32)
    iob = jnp.minimum(iob, nbt - 1.0)
    be = jnp.sum((iob >= cumb_ex).astype(jnp.float32), axis=1,
                 keepdims=True) - 1.0
    be_ref[...] = be.astype(jnp.int32)
    nbt_ref[...] = nbt.astype(jnp.int32)


def _route(x, w_gate):
    return pl.pallas_call(
        _route_body,
        out_shape=[
            jax.ShapeDtypeStruct((N, 1), jnp.int32),
            jax.ShapeDtypeStruct((N, 1), jnp.int32),
            jax.ShapeDtypeStruct((N, 16), jnp.float32),
            jax.ShapeDtypeStruct((N, 16), jnp.float32),
            jax.ShapeDtypeStruct((NB, 1), jnp.int32),
            jax.ShapeDtypeStruct((1, 1), jnp.int32),
        ],
    )(x, w_gate)


_SC_CACHE = {}


def _dispatch_body(x_hbm, pos0_hbm, pos1_hbm, xs_hbm,
                   rows_v, idx0_v, idx1_v, sem):
    wid = lax.axis_index("s") * 2 + lax.axis_index("c")
    base = wid * TPW
    pltpu.sync_copy(x_hbm.at[pl.ds(base, TPW)], rows_v)
    pltpu.sync_copy(pos0_hbm.at[pl.ds(base, TPW)], idx0_v)
    pltpu.sync_copy(pos1_hbm.at[pl.ds(base, TPW)], idx1_v)
    c0 = pltpu.async_copy(rows_v, xs_hbm.at[idx0_v], sem)
    c1 = pltpu.async_copy(rows_v, xs_hbm.at[idx1_v], sem)
    c0.wait()
    c1.wait()


def _dispatch(x, p0f, p1f):
    if "dispatch" not in _SC_CACHE:
        mesh = plsc.VectorSubcoreMesh(core_axis_name="c", subcore_axis_name="s")
        _SC_CACHE["dispatch"] = functools.partial(
            pl.kernel, mesh=mesh,
            out_type=jax.ShapeDtypeStruct((P, D), jnp.float32),
            scratch_types=[
                pltpu.VMEM((TPW, D), jnp.float32),
                pltpu.VMEM((TPW,), jnp.int32),
                pltpu.VMEM((TPW,), jnp.int32),
                pltpu.SemaphoreType.DMA,
            ])(_dispatch_body)
    return _SC_CACHE["dispatch"](x, p0f, p1f)


def _ffn_body(be_ref, nbt_ref, xs_ref, W1_ref, b1_ref, W2_ref, b2_ref,
              pw_ref, pb_ref, zs_ref):
    b = pl.program_id(0)

    @pl.when(b < nbt_ref[0])
    def _compute():
        h = jax.nn.gelu(
            jnp.dot(xs_ref[...], W1_ref[0],
                    precision=lax.Precision.DEFAULT) + b1_ref[0])
        y = jnp.dot(h, W2_ref[0],
                    precision=lax.Precision.DEFAULT) + b2_ref[0]
        zs_ref[...] = lax.dot_general(
            y, pw_ref[...], (((1,), (1,)), ((), ())),
            precision=lax.Precision.DEFAULT) + pb_ref[...]


def _ffn(be, nbt, xs, W1, b1, W2, b2, proj_w, proj_b2d):
    grid_spec = pltpu.PrefetchScalarGridSpec(
        num_scalar_prefetch=2,
        grid=(NB,),
        in_specs=[
            pl.BlockSpec((BLK, D), lambda b, ber, nbr: (b, 0)),
            pl.BlockSpec((1, D, H), lambda b, ber, nbr: (ber[b], 0, 0)),
            pl.BlockSpec((1, 1, H), lambda b, ber, nbr: (ber[b], 0, 0)),
            pl.BlockSpec((1, H, D), lambda b, ber, nbr: (ber[b], 0, 0)),
            pl.BlockSpec((1, 1, D), lambda b, ber, nbr: (ber[b], 0, 0)),
            pl.BlockSpec((EMBED, D), lambda b, ber, nbr: (0, 0)),
            pl.BlockSpec((1, EMBED), lambda b, ber, nbr: (0, 0)),
        ],
        out_specs=pl.BlockSpec((BLK, EMBED), lambda b, ber, nbr: (b, 0)),
    )
    return pl.pallas_call(
        _ffn_body,
        grid_spec=grid_spec,
        out_shape=jax.ShapeDtypeStruct((P, EMBED), jnp.float32),
        compiler_params=pltpu.CompilerParams(
            dimension_semantics=("arbitrary",)),
    )(be, nbt, xs, W1, b1, W2, b2, proj_w, proj_b2d)


def _combine_body(zs_hbm, pos0_hbm, pos1_hbm, g0_hbm, g1_hbm, out_hbm,
                  idx0_v, idx1_v, a_v, b_v, g0_v, g1_v, sem):
    wid = lax.axis_index("s") * 2 + lax.axis_index("c")
    base = wid * TPW
    pltpu.sync_copy(pos0_hbm.at[pl.ds(base, TPW)], idx0_v)
    pltpu.sync_copy(pos1_hbm.at[pl.ds(base, TPW)], idx1_v)
    pltpu.sync_copy(g0_hbm.at[pl.ds(base, TPW)], g0_v)
    pltpu.sync_copy(g1_hbm.at[pl.ds(base, TPW)], g1_v)
    c0 = pltpu.async_copy(zs_hbm.at[idx0_v], a_v, sem)
    c1 = pltpu.async_copy(zs_hbm.at[idx1_v], b_v, sem)
    c0.wait()
    c1.wait()

    def rbody(r, c):
        gv0 = g0_v[r]
        gv1 = g1_v[r]
        for j in range(EMBED // 16):
            sl = pl.ds(j * 16, 16)
            a_v[r, sl] = gv0 * a_v[r, sl] + gv1 * b_v[r, sl]
        return c

    lax.fori_loop(0, TPW, rbody, 0)
    pltpu.sync_copy(a_v, out_hbm.at[pl.ds(base, TPW)])


def _combine(zs, p0f, p1f, g0b, g1b):
    if "combine" not in _SC_CACHE:
        mesh = plsc.VectorSubcoreMesh(core_axis_name="c", subcore_axis_name="s")
        _SC_CACHE["combine"] = functools.partial(
            pl.kernel, mesh=mesh,
            out_type=jax.ShapeDtypeStruct((N, EMBED), jnp.float32),
            scratch_types=[
                pltpu.VMEM((TPW,), jnp.int32),
                pltpu.VMEM((TPW,), jnp.int32),
                pltpu.VMEM((TPW, EMBED), jnp.float32),
                pltpu.VMEM((TPW, EMBED), jnp.float32),
                pltpu.VMEM((TPW, 16), jnp.float32),
                pltpu.VMEM((TPW, 16), jnp.float32),
                pltpu.SemaphoreType.DMA,
            ])(_combine_body)
    return _SC_CACHE["combine"](zs, p0f, p1f, g0b, g1b)


def kernel(x, w_gate, W1, b1, W2, b2, proj_w, proj_b):
    pos0, pos1, g0b, g1b, be, nbt = _route(x, w_gate)
    p0f = pos0.reshape(N)
    p1f = pos1.reshape(N)
    xs = _dispatch(x, p0f, p1f)
    zs = _ffn(be.reshape(NB), nbt.reshape(1), xs,
              W1, b1.reshape(E, 1, H), W2, b2.reshape(E, 1, D),
              proj_w, proj_b.reshape(1, EMBED))
    return _combine(zs, p0f, p1f, g0b, g1b)
